# baseline (speedup 1.0000x reference)
import functools

import jax
import jax.numpy as jnp
from jax import lax
from jax.experimental import pallas as pl
from jax.experimental.pallas import tpu as pltpu
from jax.experimental.pallas import tpu_sc as plsc

EPS = 1e-12
K_ROWS = 2


def _rsqrt_newton(x):
    i = plsc.bitcast(x, jnp.int32)
    i = jnp.int32(0x5F3759DF) - lax.shift_right_logical(i, 1)
    y = plsc.bitcast(i, jnp.float32)
    xh = x * jnp.float32(0.5)
    for _ in range(3):
        y = y * (jnp.float32(1.5) - xh * y * y)
    return y


def _build_sc_kernel(N, H, NC, NS, LANES):
    NW = NC * NS
    CHUNK = K_ROWS * 128
    RPW = (N // 128) // NW
    NCHUNK = RPW // K_ROWS
    GROUPS = CHUNK // LANES
    JCOLS = H // LANES

    mesh = plsc.VectorSubcoreMesh(core_axis_name="c", subcore_axis_name="s")

    @functools.partial(
        pl.kernel,
        out_type=jax.ShapeDtypeStruct((N, H), jnp.float32),
        mesh=mesh,
        compiler_params=pltpu.CompilerParams(needs_layout_passes=False),
        scratch_types=[
            pltpu.VMEM((K_ROWS, 128), jnp.int32),
            pltpu.VMEM((CHUNK,), jnp.int32),
            pltpu.VMEM((CHUNK, H), jnp.float32),
            pltpu.VMEM((2, H), jnp.float32),
            pltpu.VMEM((H,), jnp.float32),
            pltpu.VMEM((H,), jnp.float32),
            pltpu.VMEM((3 * 16,), jnp.float32),
            pltpu.SemaphoreType.DMA,
        ],
    )
    def behrt_sc(ids_hbm, tt_hbm, word_hbm, seg_hbm, gamma_hbm, beta_hbm,
                 out_hbm, idx_v, tt_v, rows_v, seg_v, gamma_v, beta_v, ac_v,
                 sem):
        wid = lax.axis_index("s") * NC + lax.axis_index("c")
        row0 = wid * RPW
        pltpu.sync_copy(seg_hbm, seg_v)
        pltpu.sync_copy(gamma_hbm, gamma_v)
        pltpu.sync_copy(beta_hbm, beta_v)
        iota = lax.iota(jnp.int32, LANES)
        zero = jnp.zeros((LANES,), jnp.float32)

        def do_chunk(c, carry):
            r = row0 + c * K_ROWS
            tok0 = r * 128
            pltpu.sync_copy(ids_hbm.at[pl.ds(r, K_ROWS)], idx_v)
            pltpu.sync_copy(tt_hbm.at[pl.ds(tok0, CHUNK)], tt_v)
            for j in range(K_ROWS):
                pltpu.async_copy(word_hbm.at[idx_v.at[j]],
                                 rows_v.at[pl.ds(j * 128, 128)], sem).wait()

            def do_group(g, gcarry):
                tok16 = g * LANES + iota
                tt16 = plsc.load_gather(tt_v, [tok16])

                def p1(d, st):
                    s, ss = st
                    dv = jnp.full((LANES,), d, jnp.int32)
                    w = plsc.load_gather(rows_v, [tok16, dv])
                    sg = plsc.load_gather(seg_v, [tt16, dv])
                    y = w + sg
                    plsc.store_scatter(rows_v, [tok16, dv], y)
                    return (s + y, ss + y * y)

                s, ss = lax.fori_loop(0, H, p1, (zero, zero))
                mean = s * jnp.float32(1.0 / H)
                var = ss * jnp.float32(1.0 / H) - mean * mean
                rstd = _rsqrt_newton(var + jnp.float32(EPS))
                ac_v[pl.ds(LANES, LANES)] = rstd
                ac_v[pl.ds(2 * LANES, LANES)] = -mean * rstd

                gam = [gamma_v[pl.ds(j2 * LANES, LANES)] for j2 in range(JCOLS)]
                bet = [beta_v[pl.ds(j2 * LANES, LANES)] for j2 in range(JCOLS)]
                for t in range(LANES):
                    at = plsc.load_gather(ac_v, [jnp.full((LANES,), t + LANES, jnp.int32)])
                    ct = plsc.load_gather(ac_v, [jnp.full((LANES,), t + 2 * LANES, jnp.int32)])
                    trow = jnp.full((LANES,), g * LANES + t, jnp.int32)
                    for j2 in range(JCOLS):
                        col = j2 * LANES + iota
                        y = plsc.load_gather(rows_v, [trow, col])
                        o = (y * at + ct) * gam[j2] + bet[j2]
                        plsc.store_scatter(rows_v, [trow, col], o)
                return gcarry

            lax.fori_loop(0, GROUPS, do_group, 0)
            pltpu.sync_copy(rows_v, out_hbm.at[pl.ds(tok0, CHUNK)])
            return carry

        lax.fori_loop(0, NCHUNK, do_chunk, 0)

    return behrt_sc


def kernel(input_ids, token_type_ids, word_table, segment_table, ln_gamma,
           ln_beta):
    B, L = input_ids.shape
    V, H = word_table.shape
    N = B * L
    info = plsc.get_sparse_core_info()
    NC, NS, LANES = info.num_cores, info.num_subcores, info.num_lanes

    ids2d = input_ids.reshape(N // 128, 128).astype(jnp.int32)
    tt_flat = token_type_ids.reshape(N).astype(jnp.int32)

    sc = _build_sc_kernel(N, H, NC, NS, LANES)
    out = sc(ids2d, tt_flat, word_table, segment_table, ln_gamma, ln_beta)
    return out.reshape(B, L, H)


# speedup vs baseline: 1.0585x; 1.0585x over previous
import functools

import jax
import jax.numpy as jnp
from jax import lax
from jax.experimental import pallas as pl
from jax.experimental.pallas import tpu as pltpu
from jax.experimental.pallas import tpu_sc as plsc

EPS = 1e-12
CHUNK = 128
NBUF = 4
UNROLL = 8


def _rsqrt_newton(x):
    i = plsc.bitcast(x, jnp.int32)
    i = jnp.int32(0x5F3759DF) - lax.shift_right_logical(i, 1)
    y = plsc.bitcast(i, jnp.float32)
    xh = x * jnp.float32(0.5)
    for _ in range(3):
        y = y * (jnp.float32(1.5) - xh * y * y)
    return y


def _build_sc_kernel(N, H, NC, NS, LANES):
    NW = NC * NS
    RPW = (N // CHUNK) // NW
    NCHUNK = RPW
    GROUPS = CHUNK // LANES
    JCOLS = H // LANES

    mesh = plsc.VectorSubcoreMesh(core_axis_name="c", subcore_axis_name="s")

    scratch = (
        [pltpu.VMEM((CHUNK,), jnp.int32) for _ in range(NBUF)]
        + [pltpu.VMEM((CHUNK,), jnp.int32) for _ in range(NBUF)]
        + [pltpu.VMEM((CHUNK, H), jnp.float32) for _ in range(NBUF)]
        + [
            pltpu.VMEM((2, H), jnp.float32),
            pltpu.VMEM((H,), jnp.float32),
            pltpu.VMEM((H,), jnp.float32),
            pltpu.VMEM((3 * 16,), jnp.float32),
            pltpu.SemaphoreType.DMA((NBUF,)),
            pltpu.SemaphoreType.DMA((NBUF,)),
            pltpu.SemaphoreType.DMA((NBUF,)),
        ]
    )

    @functools.partial(
        pl.kernel,
        out_type=jax.ShapeDtypeStruct((N, H), jnp.float32),
        mesh=mesh,
        compiler_params=pltpu.CompilerParams(needs_layout_passes=False),
        scratch_types=scratch,
    )
    def behrt_sc(ids_hbm, tt_hbm, word_hbm, seg_hbm, gamma_hbm, beta_hbm,
                 out_hbm, *refs):
        idx_b = refs[0:NBUF]
        tt_b = refs[NBUF:2 * NBUF]
        rows_b = refs[2 * NBUF:3 * NBUF]
        seg_v, gamma_v, beta_v, ac_v, sem_in, sem_g, sem_out = refs[3 * NBUF:]

        wid = lax.axis_index("s") * NC + lax.axis_index("c")
        row0 = wid * RPW
        pltpu.sync_copy(seg_hbm, seg_v)
        pltpu.sync_copy(gamma_hbm, gamma_v)
        pltpu.sync_copy(beta_hbm, beta_v)
        iota = lax.iota(jnp.int32, LANES)
        zero = jnp.zeros((LANES,), jnp.float32)

        def fire_in(c, b):
            r = row0 + c
            pltpu.async_copy(ids_hbm.at[r], idx_b[b], sem_in.at[b])
            pltpu.async_copy(tt_hbm.at[r], tt_b[b], sem_in.at[b])

        def wait_in(c, b):
            r = row0 + c
            pltpu.make_async_copy(ids_hbm.at[r], idx_b[b], sem_in.at[b]).wait()
            pltpu.make_async_copy(tt_hbm.at[r], tt_b[b], sem_in.at[b]).wait()

        def fire_gather(b):
            pltpu.async_copy(word_hbm.at[idx_b[b]], rows_b[b], sem_g.at[b])

        def wait_gather(b):
            pltpu.make_async_copy(
                word_hbm.at[idx_b[b]], rows_b[b], sem_g.at[b]).wait()

        def fire_out(c, b):
            tok0 = (row0 + c) * CHUNK
            pltpu.async_copy(rows_b[b], out_hbm.at[pl.ds(tok0, CHUNK)],
                             sem_out.at[b])

        def wait_out(c, b):
            tok0 = (row0 + c) * CHUNK
            pltpu.make_async_copy(rows_b[b], out_hbm.at[pl.ds(tok0, CHUNK)],
                                  sem_out.at[b]).wait()

        def compute_chunk(rows_v, tt_v):
            def do_group(g, gcarry):
                tok16 = g * LANES + iota
                tt16 = plsc.load_gather(tt_v, [tok16])

                def p1(d8, st):
                    s0, s1, q0, q1 = st
                    base = jnp.full((LANES,), d8 * UNROLL, jnp.int32)
                    for u in range(UNROLL):
                        dv = base + u if u else base
                        w = plsc.load_gather(rows_v, [tok16, dv])
                        sg = plsc.load_gather(seg_v, [tt16, dv])
                        y = w + sg
                        plsc.store_scatter(rows_v, [tok16, dv], y)
                        if u % 2 == 0:
                            s0 = s0 + y
                            q0 = q0 + y * y
                        else:
                            s1 = s1 + y
                            q1 = q1 + y * y
                    return (s0, s1, q0, q1)

                s0, s1, q0, q1 = lax.fori_loop(
                    0, H // UNROLL, p1, (zero, zero, zero, zero))
                s = s0 + s1
                ss = q0 + q1
                mean = s * jnp.float32(1.0 / H)
                var = ss * jnp.float32(1.0 / H) - mean * mean
                rstd = _rsqrt_newton(var + jnp.float32(EPS))
                ac_v[pl.ds(LANES, LANES)] = rstd
                ac_v[pl.ds(2 * LANES, LANES)] = -mean * rstd

                gam = [gamma_v[pl.ds(j * LANES, LANES)] for j in range(JCOLS)]
                bet = [beta_v[pl.ds(j * LANES, LANES)] for j in range(JCOLS)]
                for t in range(LANES):
                    at = plsc.load_gather(
                        ac_v, [jnp.full((LANES,), t + LANES, jnp.int32)])
                    ct = plsc.load_gather(
                        ac_v, [jnp.full((LANES,), t + 2 * LANES, jnp.int32)])
                    trow = jnp.full((LANES,), g * LANES + t, jnp.int32)
                    for j in range(JCOLS):
                        col = j * LANES + iota
                        y = plsc.load_gather(rows_v, [trow, col])
                        o = (y * at + ct) * gam[j] + bet[j]
                        plsc.store_scatter(rows_v, [trow, col], o)
                return gcarry

            lax.fori_loop(0, GROUPS, do_group, 0)

        for p in range(NBUF - 1):
            fire_in(p, p)
        for p in range(NBUF - 2):
            wait_in(p, p)
            fire_gather(p)

        def loop_body(cc, carry):
            for bb in range(NBUF):
                c = cc * NBUF + bb

                @pl.when(c + NBUF - 1 < NCHUNK)
                def _():
                    fire_in(c + NBUF - 1, (bb + NBUF - 1) % NBUF)

                @pl.when(c + NBUF - 2 < NCHUNK)
                def _():
                    b2 = (bb + NBUF - 2) % NBUF
                    wait_in(c + NBUF - 2, b2)

                    @pl.when(c >= 2)
                    def _():
                        wait_out(c - 2, b2)

                    fire_gather(b2)

                wait_gather(bb)
                compute_chunk(rows_b[bb], tt_b[bb])
                fire_out(c, bb)
            return carry

        lax.fori_loop(0, NCHUNK // NBUF, loop_body, 0)
        for p in range(NBUF):
            wait_out(NCHUNK - NBUF + p, p)

    return behrt_sc


def kernel(input_ids, token_type_ids, word_table, segment_table, ln_gamma,
           ln_beta):
    B, L = input_ids.shape
    V, H = word_table.shape
    N = B * L
    info = plsc.get_sparse_core_info()
    NC, NS, LANES = info.num_cores, info.num_subcores, info.num_lanes

    ids2d = input_ids.reshape(N // CHUNK, CHUNK).astype(jnp.int32)
    tt2d = token_type_ids.reshape(N // CHUNK, CHUNK).astype(jnp.int32)

    sc = _build_sc_kernel(N, H, NC, NS, LANES)
    out = sc(ids2d, tt2d, word_table, segment_table, ln_gamma, ln_beta)
    return out.reshape(B, L, H)


# speedup vs baseline: 5.7390x; 5.4218x over previous
import functools

import jax
import jax.numpy as jnp
from jax import lax
from jax.experimental import pallas as pl
from jax.experimental.pallas import tpu as pltpu
from jax.experimental.pallas import tpu_sc as plsc

EPS = 1e-12
CHUNK = 128
NBUF = 4


def _rsqrt_newton(x):
    i = plsc.bitcast(x, jnp.int32)
    i = jnp.int32(0x5F3759DF) - lax.shift_right_logical(i, 1)
    y = plsc.bitcast(i, jnp.float32)
    xh = x * jnp.float32(0.5)
    for _ in range(3):
        y = y * (jnp.float32(1.5) - xh * y * y)
    return y


def _tree_sum(vs):
    while len(vs) > 1:
        vs = [a + b for a, b in zip(vs[::2], vs[1::2])]
    return vs[0]


def _build_sc_kernel(N, H, NC, NS, LANES):
    NW = NC * NS
    NCHUNK = (N // CHUNK) // NW
    JCOLS = H // LANES

    mesh = plsc.VectorSubcoreMesh(core_axis_name="c", subcore_axis_name="s")

    scratch = (
        [pltpu.VMEM((CHUNK,), jnp.int32) for _ in range(NBUF)]
        + [pltpu.VMEM((CHUNK,), jnp.int32) for _ in range(NBUF)]
        + [pltpu.VMEM((CHUNK, H), jnp.float32) for _ in range(NBUF)]
        + [
            pltpu.VMEM((2, H), jnp.float32),
            pltpu.VMEM((H,), jnp.float32),
            pltpu.VMEM((H,), jnp.float32),
            pltpu.SemaphoreType.DMA((NBUF,)),
            pltpu.SemaphoreType.DMA((NBUF,)),
            pltpu.SemaphoreType.DMA((NBUF,)),
        ]
    )

    @functools.partial(
        pl.kernel,
        out_type=jax.ShapeDtypeStruct((N, H), jnp.float32),
        mesh=mesh,
        compiler_params=pltpu.CompilerParams(needs_layout_passes=False),
        scratch_types=scratch,
    )
    def behrt_sc(ids_hbm, tt_hbm, word_hbm, seg_hbm, gamma_hbm, beta_hbm,
                 out_hbm, *refs):
        idx_b = refs[0:NBUF]
        tt_b = refs[NBUF:2 * NBUF]
        rows_b = refs[2 * NBUF:3 * NBUF]
        seg_v, gamma_v, beta_v, sem_in, sem_g, sem_out = refs[3 * NBUF:]

        wid = lax.axis_index("s") * NC + lax.axis_index("c")
        row0 = wid * NCHUNK
        pltpu.sync_copy(seg_hbm, seg_v)
        pltpu.sync_copy(gamma_hbm, gamma_v)
        pltpu.sync_copy(beta_hbm, beta_v)
        iota = lax.iota(jnp.int32, LANES)

        def fire_in(c, b):
            r = row0 + c
            pltpu.async_copy(ids_hbm.at[r], idx_b[b], sem_in.at[b])
            pltpu.async_copy(tt_hbm.at[r], tt_b[b], sem_in.at[b])

        def wait_in(c, b):
            r = row0 + c
            pltpu.make_async_copy(ids_hbm.at[r], idx_b[b], sem_in.at[b]).wait()
            pltpu.make_async_copy(tt_hbm.at[r], tt_b[b], sem_in.at[b]).wait()

        def fire_gather(b):
            pltpu.async_copy(word_hbm.at[idx_b[b]], rows_b[b], sem_g.at[b])

        def wait_gather(b):
            pltpu.make_async_copy(
                word_hbm.at[idx_b[b]], rows_b[b], sem_g.at[b]).wait()

        def fire_out(c, b):
            tok0 = (row0 + c) * CHUNK
            pltpu.async_copy(rows_b[b], out_hbm.at[pl.ds(tok0, CHUNK)],
                             sem_out.at[b])

        def wait_out(c, b):
            tok0 = (row0 + c) * CHUNK
            pltpu.make_async_copy(rows_b[b], out_hbm.at[pl.ds(tok0, CHUNK)],
                                  sem_out.at[b]).wait()

        cols = [j * LANES + iota for j in range(JCOLS)]

        def compute_chunk(rows_v, tt_s):
            gam = [gamma_v[pl.ds(j * LANES, LANES)] for j in range(JCOLS)]
            bet = [beta_v[pl.ds(j * LANES, LANES)] for j in range(JCOLS)]
            sg0 = [seg_v[0, pl.ds(j * LANES, LANES)] for j in range(JCOLS)]
            sg1 = [seg_v[1, pl.ds(j * LANES, LANES)] for j in range(JCOLS)]
            sgd = [a - b for a, b in zip(sg1, sg0)]

            def tok_body(t, carry):
                tti = plsc.load_gather(tt_s, [jnp.full((LANES,), t, jnp.int32)])
                ttf = tti.astype(jnp.float32)
                trow = jnp.full((LANES,), t, jnp.int32)
                w = [plsc.load_gather(rows_v, [trow, cols[j]])
                     for j in range(JCOLS)]
                y = [w[j] + sg0[j] + ttf * sgd[j] for j in range(JCOLS)]
                s = _tree_sum(y)
                q = _tree_sum([v * v for v in y])
                sv = jnp.full((LANES,), jnp.sum(s))
                qv = jnp.full((LANES,), jnp.sum(q))
                mean = sv * jnp.float32(1.0 / H)
                var = qv * jnp.float32(1.0 / H) - mean * mean
                rstd = _rsqrt_newton(var + jnp.float32(EPS))
                cc = -mean * rstd
                for j in range(JCOLS):
                    o = (y[j] * rstd + cc) * gam[j] + bet[j]
                    plsc.store_scatter(rows_v, [trow, cols[j]], o)
                return carry

            lax.fori_loop(0, CHUNK, tok_body, 0)

        for p in range(NBUF - 1):
            fire_in(p, p)
        for p in range(NBUF - 2):
            wait_in(p, p)
            fire_gather(p)

        def loop_body(ccc, carry):
            for bb in range(NBUF):
                c = ccc * NBUF + bb

                @pl.when(c + NBUF - 1 < NCHUNK)
                def _():
                    fire_in(c + NBUF - 1, (bb + NBUF - 1) % NBUF)

                @pl.when(c + NBUF - 2 < NCHUNK)
                def _():
                    b2 = (bb + NBUF - 2) % NBUF
                    wait_in(c + NBUF - 2, b2)

                    @pl.when(c >= 2)
                    def _():
                        wait_out(c - 2, b2)

                    fire_gather(b2)

                wait_gather(bb)
                compute_chunk(rows_b[bb], tt_b[bb])
                fire_out(c, bb)
            return carry

        lax.fori_loop(0, NCHUNK // NBUF, loop_body, 0)
        for p in range(NBUF):
            wait_out(NCHUNK - NBUF + p, p)

    return behrt_sc


def kernel(input_ids, token_type_ids, word_table, segment_table, ln_gamma,
           ln_beta):
    B, L = input_ids.shape
    V, H = word_table.shape
    N = B * L
    info = plsc.get_sparse_core_info()
    NC, NS, LANES = info.num_cores, info.num_subcores, info.num_lanes

    ids2d = input_ids.reshape(N // CHUNK, CHUNK).astype(jnp.int32)
    tt2d = token_type_ids.reshape(N // CHUNK, CHUNK).astype(jnp.int32)

    sc = _build_sc_kernel(N, H, NC, NS, LANES)
    out = sc(ids2d, tt2d, word_table, segment_table, ln_gamma, ln_beta)
    return out.reshape(B, L, H)


# speedup vs baseline: 7.0642x; 1.2309x over previous
import functools

import jax
import jax.numpy as jnp
from jax import lax
from jax.experimental import pallas as pl
from jax.experimental.pallas import tpu as pltpu
from jax.experimental.pallas import tpu_sc as plsc

EPS = 1e-12
CHUNK = 128
NBUF = 4


def _rsqrt_newton(x):
    i = plsc.bitcast(x, jnp.int32)
    i = jnp.int32(0x5F3759DF) - lax.shift_right_logical(i, 1)
    y = plsc.bitcast(i, jnp.float32)
    xh = x * jnp.float32(0.5)
    for _ in range(3):
        y = y * (jnp.float32(1.5) - xh * y * y)
    return y


def _tree_sum(vs):
    while len(vs) > 1:
        vs = [a + b for a, b in zip(vs[::2], vs[1::2])]
    return vs[0]


def _build_sc_kernel(N, H, NC, NS, LANES):
    NW = NC * NS
    NCHUNK = (N // CHUNK) // NW
    JCOLS = H // LANES

    mesh = plsc.VectorSubcoreMesh(core_axis_name="c", subcore_axis_name="s")

    scratch = (
        [pltpu.VMEM((CHUNK,), jnp.int32) for _ in range(NBUF)]
        + [pltpu.VMEM((CHUNK,), jnp.int32) for _ in range(NBUF)]
        + [pltpu.VMEM((CHUNK, H), jnp.float32) for _ in range(NBUF)]
        + [
            pltpu.VMEM((2, H), jnp.float32),
            pltpu.VMEM((H,), jnp.float32),
            pltpu.VMEM((H,), jnp.float32),
            pltpu.SemaphoreType.DMA((NBUF,)),
            pltpu.SemaphoreType.DMA((NBUF,)),
            pltpu.SemaphoreType.DMA((NBUF,)),
        ]
    )

    @functools.partial(
        pl.kernel,
        out_type=jax.ShapeDtypeStruct((N, H), jnp.float32),
        mesh=mesh,
        compiler_params=pltpu.CompilerParams(needs_layout_passes=False),
        scratch_types=scratch,
    )
    def behrt_sc(ids_hbm, tt_hbm, word_hbm, seg_hbm, gamma_hbm, beta_hbm,
                 out_hbm, *refs):
        idx_b = refs[0:NBUF]
        tt_b = refs[NBUF:2 * NBUF]
        rows_b = refs[2 * NBUF:3 * NBUF]
        seg_v, gamma_v, beta_v, sem_in, sem_g, sem_out = refs[3 * NBUF:]

        wid = lax.axis_index("s") * NC + lax.axis_index("c")
        row0 = wid * NCHUNK
        pltpu.sync_copy(seg_hbm, seg_v)
        pltpu.sync_copy(gamma_hbm, gamma_v)
        pltpu.sync_copy(beta_hbm, beta_v)
        iota = lax.iota(jnp.int32, LANES)

        def fire_in(c, b):
            r = row0 + c
            pltpu.async_copy(ids_hbm.at[r], idx_b[b], sem_in.at[b])
            pltpu.async_copy(tt_hbm.at[r], tt_b[b], sem_in.at[b])

        def wait_in(c, b):
            r = row0 + c
            pltpu.make_async_copy(ids_hbm.at[r], idx_b[b], sem_in.at[b]).wait()
            pltpu.make_async_copy(tt_hbm.at[r], tt_b[b], sem_in.at[b]).wait()

        def fire_gather(b):
            pltpu.async_copy(word_hbm.at[idx_b[b]], rows_b[b], sem_g.at[b])

        def wait_gather(b):
            pltpu.make_async_copy(
                word_hbm.at[idx_b[b]], rows_b[b], sem_g.at[b]).wait()

        def fire_out(c, b):
            tok0 = (row0 + c) * CHUNK
            pltpu.async_copy(rows_b[b], out_hbm.at[pl.ds(tok0, CHUNK)],
                             sem_out.at[b])

        def wait_out(c, b):
            tok0 = (row0 + c) * CHUNK
            pltpu.make_async_copy(rows_b[b], out_hbm.at[pl.ds(tok0, CHUNK)],
                                  sem_out.at[b]).wait()

        cols = [j * LANES + iota for j in range(JCOLS)]

        def compute_chunk(rows_v, tt_s):
            gam = [gamma_v[pl.ds(j * LANES, LANES)] for j in range(JCOLS)]
            bet = [beta_v[pl.ds(j * LANES, LANES)] for j in range(JCOLS)]
            sg0 = [seg_v[0, pl.ds(j * LANES, LANES)] for j in range(JCOLS)]
            sg1 = [seg_v[1, pl.ds(j * LANES, LANES)] for j in range(JCOLS)]
            sgd = [a - b for a, b in zip(sg1, sg0)]

            @plsc.parallel_loop(0, CHUNK, step=1, unroll=4)
            def tok_body(t):
                tti = plsc.load_gather(tt_s, [jnp.full((LANES,), t, jnp.int32)])
                ttf = tti.astype(jnp.float32)
                trow = jnp.full((LANES,), t, jnp.int32)
                w = [plsc.load_gather(rows_v, [trow, cols[j]])
                     for j in range(JCOLS)]
                y = [w[j] + sg0[j] + ttf * sgd[j] for j in range(JCOLS)]
                s = _tree_sum(y)
                q = _tree_sum([v * v for v in y])
                sv = jnp.full((LANES,), jnp.sum(s))
                qv = jnp.full((LANES,), jnp.sum(q))
                mean = sv * jnp.float32(1.0 / H)
                var = qv * jnp.float32(1.0 / H) - mean * mean
                rstd = _rsqrt_newton(var + jnp.float32(EPS))
                cc = -mean * rstd
                for j in range(JCOLS):
                    o = (y[j] * rstd + cc) * gam[j] + bet[j]
                    plsc.store_scatter(rows_v, [trow, cols[j]], o)

        for p in range(NBUF - 1):
            fire_in(p, p)
        for p in range(NBUF - 2):
            wait_in(p, p)
            fire_gather(p)

        def loop_body(ccc, carry):
            for bb in range(NBUF):
                c = ccc * NBUF + bb

                @pl.when(c + NBUF - 1 < NCHUNK)
                def _():
                    fire_in(c + NBUF - 1, (bb + NBUF - 1) % NBUF)

                @pl.when(c + NBUF - 2 < NCHUNK)
                def _():
                    b2 = (bb + NBUF - 2) % NBUF
                    wait_in(c + NBUF - 2, b2)

                    @pl.when(c >= 2)
                    def _():
                        wait_out(c - 2, b2)

                    fire_gather(b2)

                wait_gather(bb)
                compute_chunk(rows_b[bb], tt_b[bb])
                fire_out(c, bb)
            return carry

        lax.fori_loop(0, NCHUNK // NBUF, loop_body, 0)
        for p in range(NBUF):
            wait_out(NCHUNK - NBUF + p, p)

    return behrt_sc


def kernel(input_ids, token_type_ids, word_table, segment_table, ln_gamma,
           ln_beta):
    B, L = input_ids.shape
    V, H = word_table.shape
    N = B * L
    info = plsc.get_sparse_core_info()
    NC, NS, LANES = info.num_cores, info.num_subcores, info.num_lanes

    ids2d = input_ids.reshape(N // CHUNK, CHUNK).astype(jnp.int32)
    tt2d = token_type_ids.reshape(N // CHUNK, CHUNK).astype(jnp.int32)

    sc = _build_sc_kernel(N, H, NC, NS, LANES)
    out = sc(ids2d, tt2d, word_table, segment_table, ln_gamma, ln_beta)
    return out.reshape(B, L, H)


# speedup vs baseline: 8.3225x; 1.1781x over previous
import functools

import jax
import jax.numpy as jnp
from jax import lax
from jax.experimental import pallas as pl
from jax.experimental.pallas import tpu as pltpu
from jax.experimental.pallas import tpu_sc as plsc

EPS = 1e-12
CHUNK = 128
NBUF = 4


def _rsqrt_newton(x):
    i = plsc.bitcast(x, jnp.int32)
    i = jnp.int32(0x5F3759DF) - lax.shift_right_logical(i, 1)
    y = plsc.bitcast(i, jnp.float32)
    xh = x * jnp.float32(0.5)
    for _ in range(2):
        y = y * (jnp.float32(1.5) - xh * y * y)
    return y


def _tree_sum(vs):
    while len(vs) > 1:
        vs = [a + b for a, b in zip(vs[::2], vs[1::2])]
    return vs[0]


def _build_sc_kernel(N, H, NC, NS, LANES):
    NW = NC * NS
    NCHUNK = (N // CHUNK) // NW
    JCOLS = H // LANES

    mesh = plsc.VectorSubcoreMesh(core_axis_name="c", subcore_axis_name="s")

    scratch = (
        [pltpu.VMEM((CHUNK,), jnp.int32) for _ in range(NBUF)]
        + [pltpu.VMEM((CHUNK,), jnp.int32) for _ in range(NBUF)]
        + [pltpu.VMEM((CHUNK, H), jnp.float32) for _ in range(NBUF)]
        + [
            pltpu.VMEM((2, H), jnp.float32),
            pltpu.VMEM((H,), jnp.float32),
            pltpu.VMEM((H,), jnp.float32),
            pltpu.SemaphoreType.DMA((NBUF,)),
            pltpu.SemaphoreType.DMA((NBUF,)),
            pltpu.SemaphoreType.DMA((NBUF,)),
        ]
    )

    @functools.partial(
        pl.kernel,
        out_type=jax.ShapeDtypeStruct((N, H), jnp.float32),
        mesh=mesh,
        compiler_params=pltpu.CompilerParams(needs_layout_passes=False),
        scratch_types=scratch,
    )
    def behrt_sc(ids_hbm, tt_hbm, word_hbm, seg_hbm, gamma_hbm, beta_hbm,
                 out_hbm, *refs):
        idx_b = refs[0:NBUF]
        tt_b = refs[NBUF:2 * NBUF]
        rows_b = refs[2 * NBUF:3 * NBUF]
        seg_v, gamma_v, beta_v, sem_in, sem_g, sem_out = refs[3 * NBUF:]

        wid = lax.axis_index("s") * NC + lax.axis_index("c")
        row0 = wid * NCHUNK
        pltpu.sync_copy(seg_hbm, seg_v)
        pltpu.sync_copy(gamma_hbm, gamma_v)
        pltpu.sync_copy(beta_hbm, beta_v)
        iota = lax.iota(jnp.int32, LANES)

        def fire_in(c, b):
            r = row0 + c
            pltpu.async_copy(ids_hbm.at[r], idx_b[b], sem_in.at[b])
            pltpu.async_copy(tt_hbm.at[r], tt_b[b], sem_in.at[b])

        def wait_in(c, b):
            r = row0 + c
            pltpu.make_async_copy(ids_hbm.at[r], idx_b[b], sem_in.at[b]).wait()
            pltpu.make_async_copy(tt_hbm.at[r], tt_b[b], sem_in.at[b]).wait()

        def fire_gather(b):
            pltpu.async_copy(word_hbm.at[idx_b[b]], rows_b[b], sem_g.at[b])

        def wait_gather(b):
            pltpu.make_async_copy(
                word_hbm.at[idx_b[b]], rows_b[b], sem_g.at[b]).wait()

        def fire_out(c, b):
            tok0 = (row0 + c) * CHUNK
            pltpu.async_copy(rows_b[b], out_hbm.at[pl.ds(tok0, CHUNK)],
                             sem_out.at[b])

        def wait_out(c, b):
            tok0 = (row0 + c) * CHUNK
            pltpu.make_async_copy(rows_b[b], out_hbm.at[pl.ds(tok0, CHUNK)],
                                  sem_out.at[b]).wait()

        cols = [j * LANES + iota for j in range(JCOLS)]

        def compute_chunk(rows_v, tt_s):
            gam = [gamma_v[pl.ds(j * LANES, LANES)] for j in range(JCOLS)]
            bet = [beta_v[pl.ds(j * LANES, LANES)] for j in range(JCOLS)]
            sg0 = [seg_v[0, pl.ds(j * LANES, LANES)] for j in range(JCOLS)]
            sg1 = [seg_v[1, pl.ds(j * LANES, LANES)] for j in range(JCOLS)]
            sgd = [a - b for a, b in zip(sg1, sg0)]

            @plsc.parallel_loop(0, CHUNK, step=1, unroll=8)
            def tok_body(t):
                tti = plsc.load_gather(tt_s, [jnp.full((LANES,), t, jnp.int32)])
                ttf = tti.astype(jnp.float32)
                trow = jnp.full((LANES,), t, jnp.int32)
                w = [plsc.load_gather(rows_v, [trow, cols[j]])
                     for j in range(JCOLS)]
                y = [w[j] + sg0[j] + ttf * sgd[j] for j in range(JCOLS)]
                s = _tree_sum(y)
                q = _tree_sum([v * v for v in y])
                sv = jnp.full((LANES,), jnp.sum(s))
                qv = jnp.full((LANES,), jnp.sum(q))
                mean = sv * jnp.float32(1.0 / H)
                var = qv * jnp.float32(1.0 / H) - mean * mean
                rstd = _rsqrt_newton(var + jnp.float32(EPS))
                cc = -mean * rstd
                for j in range(JCOLS):
                    o = (y[j] * rstd + cc) * gam[j] + bet[j]
                    plsc.store_scatter(rows_v, [trow, cols[j]], o)

        for p in range(NBUF - 1):
            fire_in(p, p)
        for p in range(NBUF - 2):
            wait_in(p, p)
            fire_gather(p)

        def loop_body(ccc, carry):
            for bb in range(NBUF):
                c = ccc * NBUF + bb

                @pl.when(c + NBUF - 1 < NCHUNK)
                def _():
                    fire_in(c + NBUF - 1, (bb + NBUF - 1) % NBUF)

                @pl.when(c + NBUF - 2 < NCHUNK)
                def _():
                    b2 = (bb + NBUF - 2) % NBUF
                    wait_in(c + NBUF - 2, b2)

                    @pl.when(c >= 2)
                    def _():
                        wait_out(c - 2, b2)

                    fire_gather(b2)

                wait_gather(bb)
                compute_chunk(rows_b[bb], tt_b[bb])
                fire_out(c, bb)
            return carry

        lax.fori_loop(0, NCHUNK // NBUF, loop_body, 0)
        for p in range(NBUF):
            wait_out(NCHUNK - NBUF + p, p)

    return behrt_sc


def kernel(input_ids, token_type_ids, word_table, segment_table, ln_gamma,
           ln_beta):
    B, L = input_ids.shape
    V, H = word_table.shape
    N = B * L
    info = plsc.get_sparse_core_info()
    NC, NS, LANES = info.num_cores, info.num_subcores, info.num_lanes

    ids2d = input_ids.reshape(N // CHUNK, CHUNK).astype(jnp.int32)
    tt2d = token_type_ids.reshape(N // CHUNK, CHUNK).astype(jnp.int32)

    sc = _build_sc_kernel(N, H, NC, NS, LANES)
    out = sc(ids2d, tt2d, word_table, segment_table, ln_gamma, ln_beta)
    return out.reshape(B, L, H)
